# Initial kernel scaffold; baseline (speedup 1.0000x reference)
import jax
import jax.numpy as jnp
from jax.experimental import pallas as pl


def kernel(points_xyz, features, W1, b1, W2, b2, Wl, bl, Wf, bf):
    raise NotImplementedError("write your pallas kernel here")


# speedup vs baseline: 9.7617x; 9.7617x over previous
import functools

import jax
import jax.numpy as jnp
from jax.experimental import pallas as pl

B = 8
N = 1024
K = 16
C_IN = 64
C_OUT = 128
C_LIFT = 64
TILE = 256
BIG = 1e9
HUGE = 3e9


def _xconv_kernel(q_ref, p_ref, pT_ref, feat_ref, W1_ref, b1_ref, W2_ref,
                  b2_ref, Wl_ref, bl_ref, Wf_ref, bf_ref, out_ref):
    q = q_ref[0]
    p_full = p_ref[0]
    pT = pT_ref[0]
    feat = feat_ref[0]

    p0 = pT[0:1, :]
    p1 = pT[1:2, :]
    p2 = pT[2:3, :]
    valid_p = (p0 != 0.0) | (p1 != 0.0) | (p2 != 0.0)
    valid_col = jnp.any(p_full != 0.0, axis=1, keepdims=True)
    valid_q = jnp.any(q != 0.0, axis=1, keepdims=True)

    d0 = q[:, 0:1] - p0
    d1 = q[:, 1:2] - p1
    d2 = q[:, 2:3] - p2
    pd = d0 * d0 + d1 * d1 + d2 * d2
    pd = jnp.where(valid_q & valid_p, pd, BIG)

    lifted = jax.nn.relu(jnp.dot(feat, Wl_ref[...],
                                 preferred_element_type=jnp.float32)
                         + bl_ref[...])
    lifted = jnp.where(valid_col, lifted, 0.0)

    payload = jnp.concatenate(
        [p_full, valid_col.astype(jnp.float32), lifted], axis=1)

    iota = jax.lax.broadcasted_iota(jnp.int32, (TILE, N), 1)
    INT_BIG = jnp.int32(2 ** 30)

    h_parts = []
    L_parts = []
    for _ in range(K):
        row_min = jnp.min(pd, axis=1, keepdims=True)
        cand = jnp.where(pd <= row_min, iota, INT_BIG)
        arg = jnp.min(cand, axis=1, keepdims=True)
        oh = iota == arg
        pd = jnp.where(oh, HUGE, pd)
        ohf = oh.astype(jnp.float32)
        g = jnp.dot(ohf, payload, preferred_element_type=jnp.float32)
        nbr_valid = g[:, 3:4] > 0.5
        rel = jnp.where(nbr_valid, g[:, 0:3] - q, 0.0)
        h = jax.nn.relu(jnp.dot(rel, W1_ref[...],
                                preferred_element_type=jnp.float32)
                        + b1_ref[...])
        h_parts.append(h)
        L_parts.append(g[:, 4:4 + C_LIFT])

    h_flat = jnp.concatenate(h_parts, axis=1)
    X = jnp.dot(h_flat, W2_ref[...],
                preferred_element_type=jnp.float32) + b2_ref[...]

    t_parts = []
    for i in range(K):
        acc = X[:, i * K:i * K + 1] * L_parts[0]
        for j in range(1, K):
            acc = acc + X[:, i * K + j:i * K + j + 1] * L_parts[j]
        t_parts.append(acc)
    t_flat = jnp.concatenate(t_parts, axis=1)

    final = jax.nn.relu(jnp.dot(t_flat, Wf_ref[...],
                                preferred_element_type=jnp.float32)
                        + bf_ref[...])
    out_ref[0] = jnp.where(valid_q, final, 0.0)


@jax.jit
def kernel(points_xyz, features, W1, b1, W2, b2, Wl, bl, Wf, bf):
    pT = jnp.transpose(points_xyz, (0, 2, 1))
    grid = (B, N // TILE)
    out = pl.pallas_call(
        _xconv_kernel,
        grid=grid,
        in_specs=[
            pl.BlockSpec((1, TILE, 3), lambda b, t: (b, t, 0)),
            pl.BlockSpec((1, N, 3), lambda b, t: (b, 0, 0)),
            pl.BlockSpec((1, 3, N), lambda b, t: (b, 0, 0)),
            pl.BlockSpec((1, N, C_IN), lambda b, t: (b, 0, 0)),
            pl.BlockSpec(W1.shape, lambda b, t: (0, 0)),
            pl.BlockSpec((1, K * 2), lambda b, t: (0, 0)),
            pl.BlockSpec(W2.shape, lambda b, t: (0, 0)),
            pl.BlockSpec((1, K * K), lambda b, t: (0, 0)),
            pl.BlockSpec(Wl.shape, lambda b, t: (0, 0)),
            pl.BlockSpec((1, C_LIFT), lambda b, t: (0, 0)),
            pl.BlockSpec(Wf.shape, lambda b, t: (0, 0)),
            pl.BlockSpec((1, C_OUT), lambda b, t: (0, 0)),
        ],
        out_specs=pl.BlockSpec((1, TILE, C_OUT), lambda b, t: (b, t, 0)),
        out_shape=jax.ShapeDtypeStruct((B, N, C_OUT), jnp.float32),
    )(points_xyz, points_xyz, pT, features, W1, b1.reshape(1, -1), W2,
      b2.reshape(1, -1), Wl, bl.reshape(1, -1), Wf, bf.reshape(1, -1))
    return out
